# baseline (speedup 1.0000x reference)
import jax
import jax.numpy as jnp
from jax.experimental import pallas as pl

B = 256
N = 512
PC_DIM = 3
PC_OUT = 256
STATE_DIM = 19
STATE_FEAT = 64
D_MODEL = PC_OUT + STATE_FEAT
E = 16
HID = 256
OUT = D_MODEL

NC = 64
GRID = N // NC


def _enc_body(pcn_ref, W1T, b1T, W2T, b2T, W3T, b3T, g_ref):
    x = pcn_ref[...]
    h = jnp.maximum(
        jnp.dot(W1T[...], x, preferred_element_type=jnp.float32) + b1T[...],
        0.0).astype(jnp.bfloat16)
    h = jnp.maximum(
        jnp.dot(W2T[...], h, preferred_element_type=jnp.float32) + b2T[...],
        0.0).astype(jnp.bfloat16)
    h = jnp.maximum(
        jnp.dot(W3T[...], h, preferred_element_type=jnp.float32) + b3T[...],
        0.0).astype(jnp.bfloat16)
    w = NC * B
    while w > B:
        half = w // 2
        h = jnp.maximum(h[:, :half], h[:, half:w])
        w = half
    m = h

    @pl.when(pl.program_id(0) == 0)
    def _init():
        g_ref[...] = m

    @pl.when(pl.program_id(0) > 0)
    def _acc():
        g_ref[...] = jnp.maximum(g_ref[...], m)


def _moe_body(g_ref, ap_ref, WpT, bpT, Ws1T, bs1T, Ws2T, bs2T, WrT, brT,
              We1_ref, be1T_ref, We2_ref, be2T_ref,
              out_ref, load_ref, ent_ref,
              xTb_ref, gate_ref, acc_ref):
    e = pl.program_id(0)

    @pl.when(e == 0)
    def _route():
        gT = g_ref[...]
        pcfT = jnp.dot(WpT[...], gT, preferred_element_type=jnp.float32) + bpT[...]
        apT = ap_ref[...]
        sT = jnp.maximum(
            jnp.dot(Ws1T[...], apT, preferred_element_type=jnp.float32)
            + bs1T[...], 0.0)
        sT = jnp.dot(Ws2T[...], sT, preferred_element_type=jnp.float32) + bs2T[...]
        xT = jnp.concatenate([pcfT, sT], axis=0)
        xTb_ref[...] = xT.astype(jnp.bfloat16)
        acc_ref[...] = xT

        logitsT = jnp.dot(WrT[...], xT, preferred_element_type=jnp.float32) + brT[...]
        m = jnp.max(logitsT, axis=0, keepdims=True)
        ex = jnp.exp(logitsT - m)
        p = ex / jnp.sum(ex, axis=0, keepdims=True)

        eidx = jax.lax.broadcasted_iota(jnp.int32, (E, B), 0)
        m1 = jnp.max(p, axis=0, keepdims=True)
        i1 = jnp.min(jnp.where(p == m1, eidx, E), axis=0, keepdims=True)
        mask1 = eidx == i1
        pm = jnp.where(mask1, -jnp.inf, p)
        m2 = jnp.max(pm, axis=0, keepdims=True)
        i2 = jnp.min(jnp.where(pm == m2, eidx, E), axis=0, keepdims=True)
        mask2 = eidx == i2
        sw = m1 + m2 + 1e-9
        gate_ref[...] = (jnp.where(mask1, m1 / sw, 0.0)
                         + jnp.where(mask2, m2 / sw, 0.0))

        disp = mask1.astype(jnp.float32) + mask2.astype(jnp.float32)
        f_i = jnp.sum(disp, axis=1, keepdims=True) / (B * 2.0)
        P_i = jnp.sum(p, axis=1, keepdims=True) / B
        load_ref[...] = jnp.reshape(0.1 * E * jnp.sum(f_i * P_i), (1, 1))
        ent = -jnp.sum(p * jnp.log(p + 1e-9)) / B
        ent_ref[...] = jnp.reshape(-0.01 * ent, (1, 1))

    cdim = (((0,), (0,)), ((), ()))
    onehot = (jax.lax.broadcasted_iota(jnp.int32, (E, 1), 0) == e
              ).astype(jnp.float32)
    be1c = jnp.dot(be1T_ref[...], onehot, preferred_element_type=jnp.float32)
    be2c = jnp.dot(be2T_ref[...], onehot, preferred_element_type=jnp.float32)
    ehT = jnp.maximum(
        jax.lax.dot_general(We1_ref[0].astype(jnp.bfloat16), xTb_ref[...], cdim,
                            preferred_element_type=jnp.float32)
        + be1c, 0.0).astype(jnp.bfloat16)
    eyT = (jax.lax.dot_general(We2_ref[0].astype(jnp.bfloat16), ehT, cdim,
                               preferred_element_type=jnp.float32)
           + be2c)
    acc_ref[...] = acc_ref[...] + gate_ref[pl.ds(e, 1), :] * eyT

    @pl.when(e == E - 1)
    def _fin():
        out_ref[...] = acc_ref[...].T


def kernel(point_cloud, agent_pos, W1, b1, W2, b2, W3, b3, Wp, bp,
           Ws1, bs1, Ws2, bs2, Wr, br, We1, be1, We2, be2):
    bf = jnp.bfloat16
    pcn = point_cloud.astype(bf).transpose(2, 1, 0).reshape(PC_DIM, N * B)

    const = lambda shape: pl.BlockSpec(shape, lambda i: (0, 0))
    gT = pl.pallas_call(
        _enc_body,
        grid=(GRID,),
        in_specs=[
            pl.BlockSpec((PC_DIM, NC * B), lambda i: (0, i)),
            const((64, PC_DIM)), const((64, 1)),
            const((128, 64)), const((128, 1)),
            const((256, 128)), const((256, 1)),
        ],
        out_specs=pl.BlockSpec((PC_OUT, B), lambda i: (0, 0)),
        out_shape=jax.ShapeDtypeStruct((PC_OUT, B), bf),
    )(pcn, W1.T.astype(bf), b1.reshape(-1, 1),
      W2.T.astype(bf), b2.reshape(-1, 1),
      W3.T.astype(bf), b3.reshape(-1, 1))

    from jax.experimental.pallas import tpu as pltpu
    out, load, ent = pl.pallas_call(
        _moe_body,
        grid=(E,),
        in_specs=[
            const((PC_OUT, B)),
            const((STATE_DIM, B)),
            const((PC_OUT, 256)), const((PC_OUT, 1)),
            const((STATE_FEAT, STATE_DIM)), const((STATE_FEAT, 1)),
            const((STATE_FEAT, STATE_FEAT)), const((STATE_FEAT, 1)),
            const((E, D_MODEL)), const((E, 1)),
            pl.BlockSpec((1, D_MODEL, HID), lambda e: (e, 0, 0)),
            const((HID, E)),
            pl.BlockSpec((1, HID, OUT), lambda e: (e, 0, 0)),
            const((OUT, E)),
        ],
        out_specs=[
            pl.BlockSpec((B, OUT), lambda e: (0, 0)),
            pl.BlockSpec((1, 1), lambda e: (0, 0)),
            pl.BlockSpec((1, 1), lambda e: (0, 0)),
        ],
        out_shape=[
            jax.ShapeDtypeStruct((B, OUT), jnp.float32),
            jax.ShapeDtypeStruct((1, 1), jnp.float32),
            jax.ShapeDtypeStruct((1, 1), jnp.float32),
        ],
        scratch_shapes=[
            pltpu.VMEM((D_MODEL, B), bf),
            pltpu.VMEM((E, B), jnp.float32),
            pltpu.VMEM((D_MODEL, B), jnp.float32),
        ],
    )(gT, agent_pos.T, Wp.T.astype(bf), bp.reshape(-1, 1),
      Ws1.T, bs1.reshape(-1, 1), Ws2.T, bs2.reshape(-1, 1),
      Wr.T, br.reshape(-1, 1),
      We1, be1.T, We2, be2.T)
    return out, load[0, 0], ent[0, 0]


# speedup vs baseline: 1.0972x; 1.0972x over previous
import jax
import jax.numpy as jnp
from jax.experimental import pallas as pl
from jax.experimental.pallas import tpu as pltpu

B = 256
N = 512
PC_DIM = 3
PC_OUT = 256
STATE_DIM = 19
STATE_FEAT = 64
D_MODEL = PC_OUT + STATE_FEAT
E = 16
HID = 256
OUT = D_MODEL

NC = 64
GRID = N // NC
EPS = E // GRID


def _body(pcn_ref, ap_ref, W1T, b1T, W2T, b2T, W3T, b3T, WpT, bpT,
          Ws1T, bs1T, Ws2T, bs2T, WrT, brT,
          We1_ref, be1T_ref, We2_ref, be2T_ref,
          out_ref, load_ref, ent_ref,
          gmax, we1b, we2b):
    i = pl.program_id(0)

    @pl.when(i < GRID)
    def _enc():
        we1b[pl.ds(EPS * i, EPS)] = We1_ref[...].astype(jnp.bfloat16)
        we2b[pl.ds(EPS * i, EPS)] = We2_ref[...].astype(jnp.bfloat16)

        x = pcn_ref[...]
        h = jnp.maximum(
            jnp.dot(W1T[...], x, preferred_element_type=jnp.float32) + b1T[...],
            0.0).astype(jnp.bfloat16)
        h = jnp.maximum(
            jnp.dot(W2T[...], h, preferred_element_type=jnp.float32) + b2T[...],
            0.0).astype(jnp.bfloat16)
        h = jnp.maximum(
            jnp.dot(W3T[...], h, preferred_element_type=jnp.float32) + b3T[...],
            0.0).astype(jnp.bfloat16)
        w = NC * B
        while w > B:
            half = w // 2
            h = jnp.maximum(h[:, :half], h[:, half:w])
            w = half

        @pl.when(i == 0)
        def _init():
            gmax[...] = h

        @pl.when(i > 0)
        def _acc():
            gmax[...] = jnp.maximum(gmax[...], h)

    @pl.when(i == GRID)
    def _moe():
        gT = gmax[...]
        pcfT = jnp.dot(WpT[...], gT, preferred_element_type=jnp.float32) + bpT[...]
        apT = ap_ref[...]
        sT = jnp.maximum(
            jnp.dot(Ws1T[...], apT, preferred_element_type=jnp.float32)
            + bs1T[...], 0.0)
        sT = jnp.dot(Ws2T[...], sT, preferred_element_type=jnp.float32) + bs2T[...]
        xT = jnp.concatenate([pcfT, sT], axis=0)

        logitsT = jnp.dot(WrT[...], xT, preferred_element_type=jnp.float32) + brT[...]
        m = jnp.max(logitsT, axis=0, keepdims=True)
        ex = jnp.exp(logitsT - m)
        p = ex / jnp.sum(ex, axis=0, keepdims=True)

        eidx = jax.lax.broadcasted_iota(jnp.int32, (E, B), 0)
        m1 = jnp.max(p, axis=0, keepdims=True)
        i1 = jnp.min(jnp.where(p == m1, eidx, E), axis=0, keepdims=True)
        mask1 = eidx == i1
        pm = jnp.where(mask1, -jnp.inf, p)
        m2 = jnp.max(pm, axis=0, keepdims=True)
        i2 = jnp.min(jnp.where(pm == m2, eidx, E), axis=0, keepdims=True)
        mask2 = eidx == i2
        sw = m1 + m2 + 1e-9
        gateT = jnp.where(mask1, m1 / sw, 0.0) + jnp.where(mask2, m2 / sw, 0.0)

        disp = mask1.astype(jnp.float32) + mask2.astype(jnp.float32)
        f_i = jnp.sum(disp, axis=1, keepdims=True) / (B * 2.0)
        P_i = jnp.sum(p, axis=1, keepdims=True) / B
        load_ref[...] = jnp.reshape(0.1 * E * jnp.sum(f_i * P_i), (1, 1))
        ent = -jnp.sum(p * jnp.log(p + 1e-9)) / B
        ent_ref[...] = jnp.reshape(-0.01 * ent, (1, 1))

        xTb = xT.astype(jnp.bfloat16)
        acc = xT
        cdim = (((0,), (0,)), ((), ()))
        for ei in range(E):
            ehT = jnp.maximum(
                jax.lax.dot_general(we1b[ei], xTb, cdim,
                                    preferred_element_type=jnp.float32)
                + be1T_ref[:, ei:ei + 1], 0.0).astype(jnp.bfloat16)
            eyT = (jax.lax.dot_general(we2b[ei], ehT, cdim,
                                       preferred_element_type=jnp.float32)
                   + be2T_ref[:, ei:ei + 1])
            acc = acc + gateT[ei:ei + 1, :] * eyT
        out_ref[...] = acc.T


def kernel(point_cloud, agent_pos, W1, b1, W2, b2, W3, b3, Wp, bp,
           Ws1, bs1, Ws2, bs2, Wr, br, We1, be1, We2, be2):
    bf = jnp.bfloat16
    f32 = jnp.float32
    pcn = point_cloud.astype(bf).transpose(2, 1, 0).reshape(PC_DIM, N * B)

    c2 = lambda shape: pl.BlockSpec(shape, lambda i: (0, 0))
    we_spec = lambda shape: pl.BlockSpec(
        shape, lambda i: (jnp.minimum(i, GRID - 1), 0, 0))

    out, load, ent = pl.pallas_call(
        _body,
        grid=(GRID + 1,),
        in_specs=[
            pl.BlockSpec((PC_DIM, NC * B),
                         lambda i: (0, jnp.minimum(i, GRID - 1))),
            c2((STATE_DIM, B)),
            c2((64, PC_DIM)), c2((64, 1)),
            c2((128, 64)), c2((128, 1)),
            c2((256, 128)), c2((256, 1)),
            c2((PC_OUT, 256)), c2((PC_OUT, 1)),
            c2((STATE_FEAT, STATE_DIM)), c2((STATE_FEAT, 1)),
            c2((STATE_FEAT, STATE_FEAT)), c2((STATE_FEAT, 1)),
            c2((E, D_MODEL)), c2((E, 1)),
            we_spec((EPS, D_MODEL, HID)),
            c2((HID, E)),
            we_spec((EPS, HID, OUT)),
            c2((OUT, E)),
        ],
        out_specs=[
            pl.BlockSpec((B, OUT), lambda i: (0, 0)),
            pl.BlockSpec((1, 1), lambda i: (0, 0)),
            pl.BlockSpec((1, 1), lambda i: (0, 0)),
        ],
        out_shape=[
            jax.ShapeDtypeStruct((B, OUT), f32),
            jax.ShapeDtypeStruct((1, 1), f32),
            jax.ShapeDtypeStruct((1, 1), f32),
        ],
        scratch_shapes=[
            pltpu.VMEM((PC_OUT, B), bf),
            pltpu.VMEM((E, D_MODEL, HID), bf),
            pltpu.VMEM((E, HID, OUT), bf),
        ],
    )(pcn, agent_pos.T,
      W1.T.astype(bf), b1.reshape(-1, 1),
      W2.T.astype(bf), b2.reshape(-1, 1),
      W3.T.astype(bf), b3.reshape(-1, 1),
      Wp.T.astype(bf), bp.reshape(-1, 1),
      Ws1.T, bs1.reshape(-1, 1),
      Ws2.T, bs2.reshape(-1, 1),
      Wr.T, br.reshape(-1, 1),
      We1, be1.T, We2, be2.T)
    return out, load[0, 0], ent[0, 0]


# speedup vs baseline: 1.1813x; 1.0767x over previous
import jax
import jax.numpy as jnp
from jax.experimental import pallas as pl

B = 256
N = 512
PC_DIM = 3
PC_OUT = 256
STATE_DIM = 19
STATE_FEAT = 64
D_MODEL = PC_OUT + STATE_FEAT
E = 16
HID = 256
OUT = D_MODEL

NC = 64
GRID = N // NC


def _enc_body(pcn_ref, W1T, b1T, W2T, b2T, W3T, b3T, g_ref):
    x = pcn_ref[...]
    h = jnp.maximum(
        jnp.dot(W1T[...], x, preferred_element_type=jnp.float32) + b1T[...],
        0.0).astype(jnp.bfloat16)
    h = jnp.maximum(
        jnp.dot(W2T[...], h, preferred_element_type=jnp.float32) + b2T[...],
        0.0).astype(jnp.bfloat16)
    h = jnp.maximum(
        jnp.dot(W3T[...], h, preferred_element_type=jnp.float32) + b3T[...],
        0.0).astype(jnp.bfloat16)
    w = NC * B
    while w > B:
        half = w // 2
        h = jnp.maximum(h[:, :half], h[:, half:w])
        w = half
    m = h

    @pl.when(pl.program_id(0) == 0)
    def _init():
        g_ref[...] = m

    @pl.when(pl.program_id(0) > 0)
    def _acc():
        g_ref[...] = jnp.maximum(g_ref[...], m)


def _moe_body(g_ref, ap_ref, WpT, bpT, Ws1T, bs1T, Ws2T, bs2T, WrT, brT,
              We1_ref, be1T_ref, We2_ref, be2T_ref,
              out_ref, load_ref, ent_ref):
    gT = g_ref[...]
    pcfT = jnp.dot(WpT[...], gT, preferred_element_type=jnp.float32) + bpT[...]
    apT = ap_ref[...]
    sT = jnp.maximum(
        jnp.dot(Ws1T[...], apT, preferred_element_type=jnp.float32) + bs1T[...], 0.0)
    sT = jnp.dot(Ws2T[...], sT, preferred_element_type=jnp.float32) + bs2T[...]
    xT = jnp.concatenate([pcfT, sT], axis=0)

    logitsT = jnp.dot(WrT[...], xT, preferred_element_type=jnp.float32) + brT[...]
    m = jnp.max(logitsT, axis=0, keepdims=True)
    ex = jnp.exp(logitsT - m)
    p = ex / jnp.sum(ex, axis=0, keepdims=True)

    eidx = jax.lax.broadcasted_iota(jnp.int32, (E, B), 0)
    m1 = jnp.max(p, axis=0, keepdims=True)
    i1 = jnp.min(jnp.where(p == m1, eidx, E), axis=0, keepdims=True)
    mask1 = eidx == i1
    pm = jnp.where(mask1, -jnp.inf, p)
    m2 = jnp.max(pm, axis=0, keepdims=True)
    i2 = jnp.min(jnp.where(pm == m2, eidx, E), axis=0, keepdims=True)
    mask2 = eidx == i2
    sw = m1 + m2 + 1e-9
    gateT = jnp.where(mask1, m1 / sw, 0.0) + jnp.where(mask2, m2 / sw, 0.0)

    disp = mask1.astype(jnp.float32) + mask2.astype(jnp.float32)
    f_i = jnp.sum(disp, axis=1, keepdims=True) / (B * 2.0)
    P_i = jnp.sum(p, axis=1, keepdims=True) / B
    load_ref[...] = jnp.reshape(0.1 * E * jnp.sum(f_i * P_i), (1, 1))
    ent = -jnp.sum(p * jnp.log(p + 1e-9)) / B
    ent_ref[...] = jnp.reshape(-0.01 * ent, (1, 1))

    xTb = xT.astype(jnp.bfloat16)
    acc = xT
    cdim = (((0,), (0,)), ((), ()))
    for ei in range(E):
        ehT = jnp.maximum(
            jax.lax.dot_general(We1_ref[ei], xTb, cdim,
                                preferred_element_type=jnp.float32)
            + be1T_ref[:, ei:ei + 1], 0.0).astype(jnp.bfloat16)
        eyT = (jax.lax.dot_general(We2_ref[ei], ehT, cdim,
                                   preferred_element_type=jnp.float32)
               + be2T_ref[:, ei:ei + 1])
        acc = acc + gateT[ei:ei + 1, :] * eyT
    out_ref[...] = acc.T


def kernel(point_cloud, agent_pos, W1, b1, W2, b2, W3, b3, Wp, bp,
           Ws1, bs1, Ws2, bs2, Wr, br, We1, be1, We2, be2):
    bf = jnp.bfloat16
    pcn = point_cloud.astype(bf).transpose(2, 1, 0).reshape(PC_DIM, N * B)

    const = lambda shape: pl.BlockSpec(shape, lambda i: (0, 0))
    gT = pl.pallas_call(
        _enc_body,
        grid=(GRID,),
        in_specs=[
            pl.BlockSpec((PC_DIM, NC * B), lambda i: (0, i)),
            const((64, PC_DIM)), const((64, 1)),
            const((128, 64)), const((128, 1)),
            const((256, 128)), const((256, 1)),
        ],
        out_specs=pl.BlockSpec((PC_OUT, B), lambda i: (0, 0)),
        out_shape=jax.ShapeDtypeStruct((PC_OUT, B), bf),
    )(pcn, W1.T.astype(bf), b1.reshape(-1, 1),
      W2.T.astype(bf), b2.reshape(-1, 1),
      W3.T.astype(bf), b3.reshape(-1, 1))

    out, load, ent = pl.pallas_call(
        _moe_body,
        out_shape=[
            jax.ShapeDtypeStruct((B, OUT), jnp.float32),
            jax.ShapeDtypeStruct((1, 1), jnp.float32),
            jax.ShapeDtypeStruct((1, 1), jnp.float32),
        ],
    )(gT, agent_pos.T, Wp.T.astype(bf), bp.reshape(-1, 1),
      Ws1.T, bs1.reshape(-1, 1), Ws2.T, bs2.reshape(-1, 1),
      Wr.T, br.reshape(-1, 1),
      We1.astype(bf), be1.T, We2.astype(bf), be2.T)
    return out, load[0, 0], ent[0, 0]


# speedup vs baseline: 1.1864x; 1.0043x over previous
import jax
import jax.numpy as jnp
from jax.experimental import pallas as pl

B = 256
N = 512
PC_DIM = 3
PC_OUT = 256
STATE_DIM = 19
STATE_FEAT = 64
D_MODEL = PC_OUT + STATE_FEAT
E = 16
HID = 256
OUT = D_MODEL

NC = 32
GRID = N // NC


def _enc_body(pcn_ref, W1T, b1T, W2T, b2T, W3T, b3T, g_ref):
    x = pcn_ref[...]
    h = jnp.maximum(
        jnp.dot(W1T[...], x, preferred_element_type=jnp.float32) + b1T[...],
        0.0).astype(jnp.bfloat16)
    h = jnp.maximum(
        jnp.dot(W2T[...], h, preferred_element_type=jnp.float32) + b2T[...],
        0.0).astype(jnp.bfloat16)
    h = jnp.maximum(
        jnp.dot(W3T[...], h, preferred_element_type=jnp.float32) + b3T[...],
        0.0).astype(jnp.bfloat16)
    w = NC * B
    while w > B:
        half = w // 2
        h = jnp.maximum(h[:, :half], h[:, half:w])
        w = half
    m = h

    @pl.when(pl.program_id(0) == 0)
    def _init():
        g_ref[...] = m

    @pl.when(pl.program_id(0) > 0)
    def _acc():
        g_ref[...] = jnp.maximum(g_ref[...], m)


def _moe_body(g_ref, ap_ref, WpT, bpT, Ws1T, bs1T, Ws2T, bs2T, WrT, brT,
              We1_ref, be1T_ref, We2_ref, be2T_ref,
              out_ref, load_ref, ent_ref):
    gT = g_ref[...]
    pcfT = jnp.dot(WpT[...], gT, preferred_element_type=jnp.float32) + bpT[...]
    apT = ap_ref[...]
    sT = jnp.maximum(
        jnp.dot(Ws1T[...], apT, preferred_element_type=jnp.float32) + bs1T[...], 0.0)
    sT = jnp.dot(Ws2T[...], sT, preferred_element_type=jnp.float32) + bs2T[...]
    xT = jnp.concatenate([pcfT, sT], axis=0)

    logitsT = jnp.dot(WrT[...], xT, preferred_element_type=jnp.float32) + brT[...]
    m = jnp.max(logitsT, axis=0, keepdims=True)
    ex = jnp.exp(logitsT - m)
    p = ex / jnp.sum(ex, axis=0, keepdims=True)

    eidx = jax.lax.broadcasted_iota(jnp.int32, (E, B), 0)
    m1 = jnp.max(p, axis=0, keepdims=True)
    i1 = jnp.min(jnp.where(p == m1, eidx, E), axis=0, keepdims=True)
    mask1 = eidx == i1
    pm = jnp.where(mask1, -jnp.inf, p)
    m2 = jnp.max(pm, axis=0, keepdims=True)
    i2 = jnp.min(jnp.where(pm == m2, eidx, E), axis=0, keepdims=True)
    mask2 = eidx == i2
    sw = m1 + m2 + 1e-9
    gateT = jnp.where(mask1, m1 / sw, 0.0) + jnp.where(mask2, m2 / sw, 0.0)

    disp = mask1.astype(jnp.float32) + mask2.astype(jnp.float32)
    f_i = jnp.sum(disp, axis=1, keepdims=True) / (B * 2.0)
    P_i = jnp.sum(p, axis=1, keepdims=True) / B
    load_ref[...] = jnp.reshape(0.1 * E * jnp.sum(f_i * P_i), (1, 1))
    ent = -jnp.sum(p * jnp.log(p + 1e-9)) / B
    ent_ref[...] = jnp.reshape(-0.01 * ent, (1, 1))

    xTb = xT.astype(jnp.bfloat16)
    acc = xT
    cdim = (((0,), (0,)), ((), ()))
    for ei in range(E):
        ehT = jnp.maximum(
            jax.lax.dot_general(We1_ref[ei], xTb, cdim,
                                preferred_element_type=jnp.float32)
            + be1T_ref[:, ei:ei + 1], 0.0).astype(jnp.bfloat16)
        eyT = (jax.lax.dot_general(We2_ref[ei], ehT, cdim,
                                   preferred_element_type=jnp.float32)
               + be2T_ref[:, ei:ei + 1])
        acc = acc + gateT[ei:ei + 1, :] * eyT
    out_ref[...] = acc


def kernel(point_cloud, agent_pos, W1, b1, W2, b2, W3, b3, Wp, bp,
           Ws1, bs1, Ws2, bs2, Wr, br, We1, be1, We2, be2):
    bf = jnp.bfloat16
    pcn = point_cloud.astype(bf).transpose(2, 1, 0).reshape(PC_DIM, N * B)

    const = lambda shape: pl.BlockSpec(shape, lambda i: (0, 0))
    gT = pl.pallas_call(
        _enc_body,
        grid=(GRID,),
        in_specs=[
            pl.BlockSpec((PC_DIM, NC * B), lambda i: (0, i)),
            const((64, PC_DIM)), const((64, 1)),
            const((128, 64)), const((128, 1)),
            const((256, 128)), const((256, 1)),
        ],
        out_specs=pl.BlockSpec((PC_OUT, B), lambda i: (0, 0)),
        out_shape=jax.ShapeDtypeStruct((PC_OUT, B), bf),
    )(pcn, W1.T.astype(bf), b1.reshape(-1, 1),
      W2.T.astype(bf), b2.reshape(-1, 1),
      W3.T.astype(bf), b3.reshape(-1, 1))

    out, load, ent = pl.pallas_call(
        _moe_body,
        out_shape=[
            jax.ShapeDtypeStruct((OUT, B), jnp.float32),
            jax.ShapeDtypeStruct((1, 1), jnp.float32),
            jax.ShapeDtypeStruct((1, 1), jnp.float32),
        ],
    )(gT, agent_pos.T, Wp.T.astype(bf), bp.reshape(-1, 1),
      Ws1.T, bs1.reshape(-1, 1), Ws2.T, bs2.reshape(-1, 1),
      Wr.T, br.reshape(-1, 1),
      We1.astype(bf), be1.T, We2.astype(bf), be2.T)
    return out.T, load[0, 0], ent[0, 0]
